# Initial kernel scaffold; baseline (speedup 1.0000x reference)
import jax
import jax.numpy as jnp
from jax.experimental import pallas as pl


def kernel(inputs, input_lens, lvl_W0, lvl_b0, lvl_W1, lvl_b1, lvl_W2, lvl_b2, attn_W, attn_b, conv_v_W, conv_v_b, bn_v_g, bn_v_b, conv_n_W, conv_n_b, bn_n_g, bn_n_b, se1_W1, se1_W2, se2_W1, se2_W2):
    raise NotImplementedError("write your pallas kernel here")


# speedup vs baseline: 12.3155x; 12.3155x over previous
import functools

import jax
import jax.numpy as jnp
from jax.experimental import pallas as pl
from jax.experimental.pallas import tpu as pltpu

B, T, D_IN, D_EMB = 16, 512, 1024, 512
VERB_C, NOUN_C = 512, 1024
NUM_VERBS, NUM_NOUNS = 10, 20

_F32 = jnp.float32


def _dot(a, b, precision=None):
    return jax.lax.dot_general(a, b, (((1,), (0,)), ((), ())),
                               preferred_element_type=_F32,
                               precision=precision)


_HI = jax.lax.Precision.HIGHEST


def _first_argmax_axis0(v, n_lanes, sentinel):
    st = jax.lax.broadcasted_iota(jnp.int32, v.shape, 0)
    m = jnp.max(v, axis=0, keepdims=True)
    return jnp.min(jnp.where(v == m, st, sentinel), axis=0, keepdims=True)


def _topk_mean(vm, kf):
    C = vm.shape[1]
    lo0 = jnp.zeros((1, C), _F32)
    hi0 = jnp.full((1, C), 1.01, _F32)

    def it(_, carry):
        lo, hi = carry
        mid = (lo + hi) * 0.5
        cnt = jnp.sum(jnp.where(vm >= mid, 1.0, 0.0), axis=0, keepdims=True)
        ge = cnt >= kf
        return jnp.where(ge, mid, lo), jnp.where(ge, hi, mid)

    lo, hi = jax.lax.fori_loop(0, 24, it, (lo0, hi0))
    incl = vm >= lo
    s = jnp.sum(jnp.where(incl, vm, 0.0), axis=0, keepdims=True)
    cge = jnp.sum(jnp.where(incl, 1.0, 0.0), axis=0, keepdims=True)
    return (s - (cge - kf) * lo) / kf


def _top_channels(vals, n, npad):
    C = vals.shape[1]
    li = jax.lax.broadcasted_iota(jnp.int32, (1, C), 1)
    pi = jax.lax.broadcasted_iota(jnp.int32, (1, npad), 1)
    sel = jnp.zeros((1, npad), jnp.int32)
    v = vals
    for j in range(n):
        m = jnp.max(v, axis=1, keepdims=True)
        idx = jnp.min(jnp.where(v == m, li, C), axis=1, keepdims=True)
        sel = jnp.where(pi == j, idx, sel)
        v = jnp.where(li == idx, -jnp.float32(jnp.inf), v)
    return sel


def _body(lens_ref, x_ref, w0_ref, b0_ref, w1_ref, b1_ref, w2_ref, b2_ref,
          aw_ref, ab_ref, wv_ref, cvb_ref, bvg_ref, bvb_ref,
          wn_ref, cnb_ref, bng_ref, bnb_ref,
          s1a_ref, s1b_ref, s2a_ref, s2b_ref,
          e1o_ref, e2o_ref, sent_ref, ilv_ref, iln_ref, tiv_ref, tin_ref,
          ev_ref, en_ref):
    i = pl.program_id(0)
    L = lens_ref[i]

    x = x_ref[0]
    e0 = _dot(x, w0_ref[...]) + b0_ref[...]
    e1 = _dot(x, w1_ref[...]) + b1_ref[...]
    e2 = _dot(x, w2_ref[...]) + b2_ref[...]
    e1o_ref[0] = e1
    e2o_ref[0] = e2

    ti = jax.lax.broadcasted_iota(jnp.int32, (T, 1), 0)
    s = _dot(e0, aw_ref[...]) + ab_ref[0, 0]
    s = jnp.where(ti >= L, -1e18, s)
    m = jnp.max(s, axis=0, keepdims=True)
    ex = jnp.exp(s - m)
    att = ex / jnp.sum(ex, axis=0, keepdims=True)
    sent_ref[0] = jnp.sum(e0 * att, axis=0, keepdims=True)

    mean1 = jnp.mean(e1, axis=0, keepdims=True)
    sc1 = jax.nn.sigmoid(_dot(jax.nn.relu(_dot(mean1, s1a_ref[...])),
                              s1b_ref[...]))
    mean2 = jnp.mean(e2, axis=0, keepdims=True)
    sc2 = jax.nn.sigmoid(_dot(jax.nn.relu(_dot(mean2, s2a_ref[...])),
                              s2b_ref[...]))

    zpad = jnp.zeros((2, D_EMB), _F32)
    e1p = jnp.concatenate([zpad, e1, zpad], axis=0)
    acc = _dot(jax.lax.slice(e1p, (0, 0), (T, D_EMB)), wv_ref[0])
    for j in range(1, 5):
        acc = acc + _dot(jax.lax.slice(e1p, (j, 0), (j + T, D_EMB)), wv_ref[j])
    lv = jax.nn.sigmoid(acc * bvg_ref[...]
                        + (cvb_ref[...] * bvg_ref[...] + bvb_ref[...]))

    lnl = jax.nn.sigmoid(_dot(e2, wn_ref[...]) * bng_ref[...]
                         + (cnb_ref[...] * bng_ref[...] + bnb_ref[...]))

    valid = ti < L
    k = jnp.clip((L + 7) // 8 - 2, 0, 62) + 2
    kf = k.astype(_F32)

    ilv = _topk_mean(jnp.where(valid, lv, -1.0), kf)
    iln = _topk_mean(jnp.where(valid, lnl, -1.0), kf)
    ilv_ref[0] = ilv
    iln_ref[0] = iln

    tiv = _top_channels(ilv, NUM_VERBS, 16)
    tin = _top_channels(iln, NUM_NOUNS, 32)
    tiv_ref[0] = tiv
    tin_ref[0] = tin

    lvf = jnp.where(valid, lv, -1e30)
    ohv = (jax.lax.broadcasted_iota(jnp.int32, (VERB_C, 16), 0)
           == tiv).astype(_F32)
    gvt = _dot(lvf, ohv, _HI)
    indv = _first_argmax_axis0(gvt, 16, T)

    lane16 = jax.lax.broadcasted_iota(jnp.int32, (1, 16), 1)
    mx = jnp.max(jnp.where(lane16 < NUM_VERBS, indv, -1))
    mn = jnp.min(jnp.where(lane16 < NUM_VERBS, indv, jnp.int32(2**30)))
    cond = jnp.logical_and(mx + 2 < L, mn > 1)

    tt = jax.lax.broadcasted_iota(jnp.int32, (T, 16), 0)
    mwin = jnp.zeros((T, 16), _F32)
    for off in range(-2, 3):
        mwin = mwin + (tt == jnp.clip(indv + off, 0, T - 1)).astype(_F32)
    msel = jnp.where(cond, mwin * 0.2, (tt == indv).astype(_F32))
    ev16 = jax.lax.dot_general(msel, e1, (((0,), (0,)), ((), ())),
                               preferred_element_type=_F32,
                               precision=_HI)
    ev_ref[0] = (ev16 * sc1)[:NUM_VERBS]

    lnf = jnp.where(valid, lnl, -1e30)
    ohn = (jax.lax.broadcasted_iota(jnp.int32, (NOUN_C, 32), 0)
           == tin).astype(_F32)
    gnt = _dot(lnf, ohn, _HI)
    tt32 = jax.lax.broadcasted_iota(jnp.int32, (T, 32), 0)
    moh = jnp.zeros((T, 32), _F32)
    v = gnt
    for _ in range(3):
        idx = _first_argmax_axis0(v, 32, T)
        hit = tt32 == idx
        moh = moh + hit.astype(_F32)
        v = jnp.where(hit, -1e30, v)
    en32 = jax.lax.dot_general(moh * (1.0 / 3.0), e2,
                               (((0,), (0,)), ((), ())),
                               preferred_element_type=_F32,
                               precision=_HI)
    en_ref[0] = (en32 * sc2)[:NUM_NOUNS]


@jax.jit
def kernel(inputs, input_lens, lvl_W0, lvl_b0, lvl_W1, lvl_b1, lvl_W2, lvl_b2,
           attn_W, attn_b, conv_v_W, conv_v_b, bn_v_g, bn_v_b,
           conv_n_W, conv_n_b, bn_n_g, bn_n_b,
           se1_W1, se1_W2, se2_W1, se2_W2):
    lens = input_lens.astype(jnp.int32)
    w0t = lvl_W0.T
    w1t = lvl_W1.T
    w2t = lvl_W2.T
    wvt = jnp.transpose(conv_v_W, (2, 1, 0))
    wnt = conv_n_W[:, :, 0].T
    awt = attn_W.T

    row = lambda a: a.reshape(1, -1)

    const = lambda shape: pl.BlockSpec(shape, lambda i: (0,) * len(shape))
    in_specs = [
        pl.BlockSpec(memory_space=pltpu.SMEM),
        pl.BlockSpec((1, T, D_IN), lambda i: (i, 0, 0)),
        const((D_IN, D_EMB)), const((1, D_EMB)),
        const((D_IN, D_EMB)), const((1, D_EMB)),
        const((D_IN, D_EMB)), const((1, D_EMB)),
        const((D_EMB, 1)), const((1, 1)),
        const((5, D_EMB, VERB_C)), const((1, VERB_C)),
        const((1, VERB_C)), const((1, VERB_C)),
        const((D_EMB, NOUN_C)), const((1, NOUN_C)),
        const((1, NOUN_C)), const((1, NOUN_C)),
        const((D_EMB, 32)), const((32, D_EMB)),
        const((D_EMB, 32)), const((32, D_EMB)),
    ]
    out_shape = [
        jax.ShapeDtypeStruct((B, T, D_EMB), _F32),
        jax.ShapeDtypeStruct((B, T, D_EMB), _F32),
        jax.ShapeDtypeStruct((B, 1, D_EMB), _F32),
        jax.ShapeDtypeStruct((B, 1, VERB_C), _F32),
        jax.ShapeDtypeStruct((B, 1, NOUN_C), _F32),
        jax.ShapeDtypeStruct((B, 1, 16), jnp.int32),
        jax.ShapeDtypeStruct((B, 1, 32), jnp.int32),
        jax.ShapeDtypeStruct((B, NUM_VERBS, D_EMB), _F32),
        jax.ShapeDtypeStruct((B, NUM_NOUNS, D_EMB), _F32),
    ]
    out_specs = [
        pl.BlockSpec((1, T, D_EMB), lambda i: (i, 0, 0)),
        pl.BlockSpec((1, T, D_EMB), lambda i: (i, 0, 0)),
        pl.BlockSpec((1, 1, D_EMB), lambda i: (i, 0, 0)),
        pl.BlockSpec((1, 1, VERB_C), lambda i: (i, 0, 0)),
        pl.BlockSpec((1, 1, NOUN_C), lambda i: (i, 0, 0)),
        pl.BlockSpec((1, 1, 16), lambda i: (i, 0, 0)),
        pl.BlockSpec((1, 1, 32), lambda i: (i, 0, 0)),
        pl.BlockSpec((1, NUM_VERBS, D_EMB), lambda i: (i, 0, 0)),
        pl.BlockSpec((1, NUM_NOUNS, D_EMB), lambda i: (i, 0, 0)),
    ]

    outs = pl.pallas_call(
        _body,
        grid=(B,),
        in_specs=in_specs,
        out_specs=out_specs,
        out_shape=out_shape,
        compiler_params=pltpu.CompilerParams(
            dimension_semantics=("arbitrary",)),
    )(lens, inputs, w0t, row(lvl_b0), w1t, row(lvl_b1), w2t, row(lvl_b2),
      awt, attn_b.reshape(1, 1), wvt, row(conv_v_b), row(bn_v_g), row(bn_v_b),
      wnt, row(conv_n_b), row(bn_n_g), row(bn_n_b),
      se1_W1.T, se1_W2.T, se2_W1.T, se2_W2.T)

    (e1o, e2o, sent, ilv, iln, tiv, tin, ev, en) = outs
    return (sent[:, 0, :], ev, en, e1o, e2o,
            ilv[:, 0, :], iln[:, 0, :],
            tiv[:, 0, :NUM_VERBS], tin[:, 0, :NUM_NOUNS])


# speedup vs baseline: 12.3167x; 1.0001x over previous
import functools

import jax
import jax.numpy as jnp
from jax.experimental import pallas as pl
from jax.experimental.pallas import tpu as pltpu

B, T, D_IN, D_EMB = 16, 512, 1024, 512
VERB_C, NOUN_C = 512, 1024
NUM_VERBS, NUM_NOUNS = 10, 20

_F32 = jnp.float32


def _dot(a, b, precision=None):
    return jax.lax.dot_general(a, b, (((1,), (0,)), ((), ())),
                               preferred_element_type=_F32,
                               precision=precision)


_HI = jax.lax.Precision.HIGHEST


def _first_argmax_axis0(v, n_lanes, sentinel):
    st = jax.lax.broadcasted_iota(jnp.int32, v.shape, 0)
    m = jnp.max(v, axis=0, keepdims=True)
    return jnp.min(jnp.where(v == m, st, sentinel), axis=0, keepdims=True)


def _topk_mean(vm, kf):
    C = vm.shape[1]
    lo0 = jnp.zeros((1, C), _F32)
    hi0 = jnp.full((1, C), 1.01, _F32)

    def it(_, carry):
        lo, hi = carry
        mid = (lo + hi) * 0.5
        cnt = jnp.sum(jnp.where(vm >= mid, 1.0, 0.0), axis=0, keepdims=True)
        ge = cnt >= kf
        return jnp.where(ge, mid, lo), jnp.where(ge, hi, mid)

    lo, hi = jax.lax.fori_loop(0, 24, it, (lo0, hi0))
    incl = vm >= lo
    s = jnp.sum(jnp.where(incl, vm, 0.0), axis=0, keepdims=True)
    cge = jnp.sum(jnp.where(incl, 1.0, 0.0), axis=0, keepdims=True)
    return (s - (cge - kf) * lo) / kf


def _top_channels(vals, n, npad):
    C = vals.shape[1]
    li = jax.lax.broadcasted_iota(jnp.int32, (1, C), 1)
    pi = jax.lax.broadcasted_iota(jnp.int32, (1, npad), 1)
    sel = jnp.zeros((1, npad), jnp.int32)
    v = vals
    for j in range(n):
        m = jnp.max(v, axis=1, keepdims=True)
        idx = jnp.min(jnp.where(v == m, li, C), axis=1, keepdims=True)
        sel = jnp.where(pi == j, idx, sel)
        v = jnp.where(li == idx, -jnp.float32(jnp.inf), v)
    return sel


def _body(lens_ref, x_ref, w0_ref, b0_ref, w1_ref, b1_ref, w2_ref, b2_ref,
          aw_ref, ab_ref, wv_ref, cvb_ref, bvg_ref, bvb_ref,
          wn_ref, cnb_ref, bng_ref, bnb_ref,
          s1a_ref, s1b_ref, s2a_ref, s2b_ref,
          e1o_ref, e2o_ref, sent_ref, ilv_ref, iln_ref, tiv_ref, tin_ref,
          ev_ref, en_ref):
    i = pl.program_id(0)
    L = lens_ref[i]

    x = x_ref[0]
    e0 = _dot(x, w0_ref[...]) + b0_ref[...]
    e1 = _dot(x, w1_ref[...]) + b1_ref[...]
    e2 = _dot(x, w2_ref[...]) + b2_ref[...]
    e1o_ref[0] = e1
    e2o_ref[0] = e2

    ti = jax.lax.broadcasted_iota(jnp.int32, (T, 1), 0)
    s = _dot(e0, aw_ref[...]) + ab_ref[0, 0]
    s = jnp.where(ti >= L, -1e18, s)
    m = jnp.max(s, axis=0, keepdims=True)
    ex = jnp.exp(s - m)
    att = ex / jnp.sum(ex, axis=0, keepdims=True)
    sent_ref[0] = jnp.sum(e0 * att, axis=0, keepdims=True)

    mean1 = jnp.mean(e1, axis=0, keepdims=True)
    sc1 = jax.nn.sigmoid(_dot(jax.nn.relu(_dot(mean1, s1a_ref[...])),
                              s1b_ref[...]))
    mean2 = jnp.mean(e2, axis=0, keepdims=True)
    sc2 = jax.nn.sigmoid(_dot(jax.nn.relu(_dot(mean2, s2a_ref[...])),
                              s2b_ref[...]))

    zpad = jnp.zeros((2, D_EMB), _F32)
    e1p = jnp.concatenate([zpad, e1, zpad], axis=0)
    acc = _dot(jax.lax.slice(e1p, (0, 0), (T, D_EMB)), wv_ref[0])
    for j in range(1, 5):
        acc = acc + _dot(jax.lax.slice(e1p, (j, 0), (j + T, D_EMB)), wv_ref[j])
    lv = jax.nn.sigmoid(acc * bvg_ref[...]
                        + (cvb_ref[...] * bvg_ref[...] + bvb_ref[...]))

    lnl = jax.nn.sigmoid(_dot(e2, wn_ref[...]) * bng_ref[...]
                         + (cnb_ref[...] * bng_ref[...] + bnb_ref[...]))

    valid = ti < L
    k = jnp.clip((L + 7) // 8 - 2, 0, 62) + 2
    kf = k.astype(_F32)

    ilv = _topk_mean(jnp.where(valid, lv, -1.0), kf)
    iln = _topk_mean(jnp.where(valid, lnl, -1.0), kf)
    ilv_ref[0] = ilv
    iln_ref[0] = iln

    tiv = _top_channels(ilv, NUM_VERBS, 16)
    tin = _top_channels(iln, NUM_NOUNS, 32)
    tiv_ref[0] = tiv
    tin_ref[0] = tin

    lvf = jnp.where(valid, lv, -1e30)
    ohv = (jax.lax.broadcasted_iota(jnp.int32, (VERB_C, 16), 0)
           == tiv).astype(_F32)
    gvt = _dot(lvf, ohv, _HI)
    indv = _first_argmax_axis0(gvt, 16, T)

    lane16 = jax.lax.broadcasted_iota(jnp.int32, (1, 16), 1)
    mx = jnp.max(jnp.where(lane16 < NUM_VERBS, indv, -1))
    mn = jnp.min(jnp.where(lane16 < NUM_VERBS, indv, jnp.int32(2**30)))
    cond = jnp.logical_and(mx + 2 < L, mn > 1)

    tt = jax.lax.broadcasted_iota(jnp.int32, (T, 16), 0)
    mwin = jnp.zeros((T, 16), _F32)
    for off in range(-2, 3):
        mwin = mwin + (tt == jnp.clip(indv + off, 0, T - 1)).astype(_F32)
    msel = jnp.where(cond, mwin * 0.2, (tt == indv).astype(_F32))
    ev16 = jax.lax.dot_general(msel, e1, (((0,), (0,)), ((), ())),
                               preferred_element_type=_F32,
                               precision=_HI)
    ev_ref[0] = (ev16 * sc1)[:NUM_VERBS]

    lnf = jnp.where(valid, lnl, -1e30)
    ohn = (jax.lax.broadcasted_iota(jnp.int32, (NOUN_C, 32), 0)
           == tin).astype(_F32)
    gnt = _dot(lnf, ohn, _HI)
    tt32 = jax.lax.broadcasted_iota(jnp.int32, (T, 32), 0)
    moh = jnp.zeros((T, 32), _F32)
    v = gnt
    for _ in range(3):
        idx = _first_argmax_axis0(v, 32, T)
        hit = tt32 == idx
        moh = moh + hit.astype(_F32)
        v = jnp.where(hit, -1e30, v)
    en32 = jax.lax.dot_general(moh * (1.0 / 3.0), e2,
                               (((0,), (0,)), ((), ())),
                               preferred_element_type=_F32,
                               precision=_HI)
    en_ref[0] = (en32 * sc2)[:NUM_NOUNS]


@jax.jit
def kernel(inputs, input_lens, lvl_W0, lvl_b0, lvl_W1, lvl_b1, lvl_W2, lvl_b2,
           attn_W, attn_b, conv_v_W, conv_v_b, bn_v_g, bn_v_b,
           conv_n_W, conv_n_b, bn_n_g, bn_n_b,
           se1_W1, se1_W2, se2_W1, se2_W2):
    lens = input_lens.astype(jnp.int32)
    w0t = lvl_W0.T
    w1t = lvl_W1.T
    w2t = lvl_W2.T
    wvt = jnp.transpose(conv_v_W, (2, 1, 0))
    wnt = conv_n_W[:, :, 0].T
    awt = attn_W.T

    row = lambda a: a.reshape(1, -1)

    const = lambda shape: pl.BlockSpec(shape, lambda i: (0,) * len(shape))
    in_specs = [
        pl.BlockSpec(memory_space=pltpu.SMEM),
        pl.BlockSpec((1, T, D_IN), lambda i: (i, 0, 0)),
        const((D_IN, D_EMB)), const((1, D_EMB)),
        const((D_IN, D_EMB)), const((1, D_EMB)),
        const((D_IN, D_EMB)), const((1, D_EMB)),
        const((D_EMB, 1)), const((1, 1)),
        const((5, D_EMB, VERB_C)), const((1, VERB_C)),
        const((1, VERB_C)), const((1, VERB_C)),
        const((D_EMB, NOUN_C)), const((1, NOUN_C)),
        const((1, NOUN_C)), const((1, NOUN_C)),
        const((D_EMB, 32)), const((32, D_EMB)),
        const((D_EMB, 32)), const((32, D_EMB)),
    ]
    out_shape = [
        jax.ShapeDtypeStruct((B, T, D_EMB), _F32),
        jax.ShapeDtypeStruct((B, T, D_EMB), _F32),
        jax.ShapeDtypeStruct((B, 1, D_EMB), _F32),
        jax.ShapeDtypeStruct((B, 1, VERB_C), _F32),
        jax.ShapeDtypeStruct((B, 1, NOUN_C), _F32),
        jax.ShapeDtypeStruct((B, 1, 16), jnp.int32),
        jax.ShapeDtypeStruct((B, 1, 32), jnp.int32),
        jax.ShapeDtypeStruct((B, NUM_VERBS, D_EMB), _F32),
        jax.ShapeDtypeStruct((B, NUM_NOUNS, D_EMB), _F32),
    ]
    out_specs = [
        pl.BlockSpec((1, T, D_EMB), lambda i: (i, 0, 0)),
        pl.BlockSpec((1, T, D_EMB), lambda i: (i, 0, 0)),
        pl.BlockSpec((1, 1, D_EMB), lambda i: (i, 0, 0)),
        pl.BlockSpec((1, 1, VERB_C), lambda i: (i, 0, 0)),
        pl.BlockSpec((1, 1, NOUN_C), lambda i: (i, 0, 0)),
        pl.BlockSpec((1, 1, 16), lambda i: (i, 0, 0)),
        pl.BlockSpec((1, 1, 32), lambda i: (i, 0, 0)),
        pl.BlockSpec((1, NUM_VERBS, D_EMB), lambda i: (i, 0, 0)),
        pl.BlockSpec((1, NUM_NOUNS, D_EMB), lambda i: (i, 0, 0)),
    ]

    outs = pl.pallas_call(
        _body,
        grid=(B,),
        in_specs=in_specs,
        out_specs=out_specs,
        out_shape=out_shape,
        compiler_params=pltpu.CompilerParams(
            dimension_semantics=("parallel",)),
    )(lens, inputs, w0t, row(lvl_b0), w1t, row(lvl_b1), w2t, row(lvl_b2),
      awt, attn_b.reshape(1, 1), wvt, row(conv_v_b), row(bn_v_g), row(bn_v_b),
      wnt, row(conv_n_b), row(bn_n_g), row(bn_n_b),
      se1_W1.T, se1_W2.T, se2_W1.T, se2_W2.T)

    (e1o, e2o, sent, ilv, iln, tiv, tin, ev, en) = outs
    return (sent[:, 0, :], ev, en, e1o, e2o,
            ilv[:, 0, :], iln[:, 0, :],
            tiv[:, 0, :NUM_VERBS], tin[:, 0, :NUM_NOUNS])
